# baseline (speedup 1.0000x reference)
import functools
import math

import numpy as np
import jax
import jax.numpy as jnp
from jax import lax
from jax.experimental import pallas as pl
from jax.experimental.pallas import tpu as pltpu
from jax.experimental.pallas import tpu_sc as plsc

B, S, D_MODEL, NHEAD = 1, 2048, 768, 12
HEAD = D_MODEL // NHEAD
N_KEYS, KNN = 256, 32
HALF = D_MODEL // 2
SCALE = 1.0 / math.sqrt(HEAD)

BQ = 256
NKB = S // BQ
BS = 512
BP = 256

SC_NC, SC_NS = 2, 16
SC_NW = SC_NC * SC_NS
TPW = (B * S) // SC_NW


def _pos_enc_rev_padded():
    inv_freq = 1.0 / (10000.0 ** (np.arange(0.0, D_MODEL, 2.0) / D_MODEL))
    pos = np.arange(-(S - 1), S, dtype=np.float64)[:, None] * inv_freq[None, :]
    pe = np.concatenate([np.sin(pos), np.cos(pos)], axis=-1)
    pe_rev = pe[::-1]
    out = np.zeros((2 * S, D_MODEL), dtype=np.float32)
    out[: 2 * S - 1] = pe_rev
    return jnp.asarray(out)


def _ln_qkv_body(x_ref, g_ref, b_ref, w_ref, qkv_ref):
    x = x_ref[...]
    m = jnp.mean(x, axis=1, keepdims=True)
    v = jnp.mean((x - m) ** 2, axis=1, keepdims=True)
    h = (x - m) * lax.rsqrt(v + 1e-5) * g_ref[...] + b_ref[...]
    qkv_ref[...] = jnp.dot(h, w_ref[...], preferred_element_type=jnp.float32)


def _ln_qkv(x, g, b, wqkv):
    return pl.pallas_call(
        _ln_qkv_body,
        grid=(S // BS,),
        in_specs=[
            pl.BlockSpec((BS, D_MODEL), lambda i: (i, 0)),
            pl.BlockSpec((1, D_MODEL), lambda i: (0, 0)),
            pl.BlockSpec((1, D_MODEL), lambda i: (0, 0)),
            pl.BlockSpec((D_MODEL, 3 * D_MODEL), lambda i: (0, 0)),
        ],
        out_specs=pl.BlockSpec((BS, 3 * D_MODEL), lambda i: (i, 0)),
        out_shape=jax.ShapeDtypeStruct((S, 3 * D_MODEL), jnp.float32),
    )(x, g, b, wqkv)


def _matmul_body(a_ref, w_ref, o_ref):
    o_ref[...] = jnp.dot(a_ref[...], w_ref[...],
                         preferred_element_type=jnp.float32)


def _rproj(pe_rev, wr):
    return pl.pallas_call(
        _matmul_body,
        grid=(2 * S // BS,),
        in_specs=[
            pl.BlockSpec((BS, D_MODEL), lambda i: (i, 0)),
            pl.BlockSpec((D_MODEL, D_MODEL), lambda i: (0, 0)),
        ],
        out_specs=pl.BlockSpec((BS, D_MODEL), lambda i: (i, 0)),
        out_shape=jax.ShapeDtypeStruct((2 * S, D_MODEL), jnp.float32),
    )(pe_rev, wr)


def _flash_body(q_ref, k_ref, v_ref, r_ref, o_ref):
    iq = pl.program_id(1)
    q = q_ref[0]
    m = jnp.full((BQ, 1), -jnp.inf, jnp.float32)
    l = jnp.zeros((BQ, 1), jnp.float32)
    acc = jnp.zeros((BQ, HEAD), jnp.float32)
    for jb in range(NKB):
        kblk = k_ref[0, pl.ds(jb * BQ, BQ), :]
        vblk = v_ref[0, pl.ds(jb * BQ, BQ), :]
        ac = lax.dot_general(q, kblk, (((1,), (1,)), ((), ())),
                             preferred_element_type=jnp.float32)
        s0 = (NKB - 1 + jb - iq) * BQ
        rband = r_ref[0, pl.ds(s0, 2 * BQ), :]
        tmp = lax.dot_general(q, rband, (((1,), (1,)), ((), ())),
                              preferred_element_type=jnp.float32)
        bd = pltpu.roll(tmp, shift=BQ + 1, axis=1,
                        stride=1, stride_axis=0)[:, :BQ]
        s = (ac + bd) * SCALE
        mnew = jnp.maximum(m, jnp.max(s, axis=1, keepdims=True))
        p = jnp.exp(s - mnew)
        corr = jnp.exp(m - mnew)
        l = l * corr + jnp.sum(p, axis=1, keepdims=True)
        acc = acc * corr + jnp.dot(p, vblk, preferred_element_type=jnp.float32)
        m = mnew
    o_ref[0] = acc / l


def _flash_attn(qh, kh, vh, rh):
    return pl.pallas_call(
        _flash_body,
        grid=(NHEAD, S // BQ),
        in_specs=[
            pl.BlockSpec((1, BQ, HEAD), lambda h, i: (h, i, 0)),
            pl.BlockSpec((1, S, HEAD), lambda h, i: (h, 0, 0)),
            pl.BlockSpec((1, S, HEAD), lambda h, i: (h, 0, 0)),
            pl.BlockSpec((1, 2 * S, HEAD), lambda h, i: (h, 0, 0)),
        ],
        out_specs=pl.BlockSpec((1, BQ, HEAD), lambda h, i: (h, i, 0)),
        out_shape=jax.ShapeDtypeStruct((NHEAD, S, HEAD), jnp.float32),
    )(qh, kh, vh, rh)


def _oproj_ln_body(a_ref, x_ref, g_ref, b_ref, wo_ref, y_ref, h_ref):
    y = jnp.dot(a_ref[...], wo_ref[...],
                preferred_element_type=jnp.float32) + x_ref[...]
    y_ref[...] = y
    m = jnp.mean(y, axis=1, keepdims=True)
    v = jnp.mean((y - m) ** 2, axis=1, keepdims=True)
    h_ref[...] = (y - m) * lax.rsqrt(v + 1e-5) * g_ref[...] + b_ref[...]


def _oproj_ln(attn, x, g, b, wo):
    return pl.pallas_call(
        _oproj_ln_body,
        grid=(S // BS,),
        in_specs=[
            pl.BlockSpec((BS, D_MODEL), lambda i: (i, 0)),
            pl.BlockSpec((BS, D_MODEL), lambda i: (i, 0)),
            pl.BlockSpec((1, D_MODEL), lambda i: (0, 0)),
            pl.BlockSpec((1, D_MODEL), lambda i: (0, 0)),
            pl.BlockSpec((D_MODEL, D_MODEL), lambda i: (0, 0)),
        ],
        out_specs=[
            pl.BlockSpec((BS, D_MODEL), lambda i: (i, 0)),
            pl.BlockSpec((BS, D_MODEL), lambda i: (i, 0)),
        ],
        out_shape=[
            jax.ShapeDtypeStruct((S, D_MODEL), jnp.float32),
            jax.ShapeDtypeStruct((S, D_MODEL), jnp.float32),
        ],
    )(attn, x, g, b, wo)


def _top_k(scores, k):
    n = scores.shape[1]
    iot = lax.broadcasted_iota(jnp.int32, scores.shape, 1)
    vals, idxs = [], []
    cur = scores
    for _ in range(k):
        mv = jnp.max(cur, axis=1, keepdims=True)
        am = jnp.min(jnp.where(cur == mv, iot, n), axis=1, keepdims=True)
        vals.append(mv)
        idxs.append(am)
        cur = jnp.where(iot == am, -jnp.inf, cur)
    return jnp.concatenate(vals, axis=1), jnp.concatenate(idxs, axis=1)


def _pkm_topk_body(h_ref, wk_ref, w_ref, idx_ref):
    s = jnp.dot(h_ref[...], wk_ref[...], preferred_element_type=jnp.float32)
    sc1, i1 = _top_k(s[:, :N_KEYS], KNN)
    sc2, i2 = _top_k(s[:, N_KEYS:], KNN)
    comb = jnp.concatenate([sc1[:, a:a + 1] + sc2 for a in range(KNN)], axis=1)
    cidx = jnp.concatenate(
        [i1[:, a:a + 1] * N_KEYS + i2 for a in range(KNN)], axis=1)
    iot = lax.broadcasted_iota(jnp.int32, comb.shape, 1)
    vals, picks = [], []
    cur = comb
    for _ in range(KNN):
        mv = jnp.max(cur, axis=1, keepdims=True)
        am = jnp.min(jnp.where(cur == mv, iot, KNN * KNN),
                     axis=1, keepdims=True)
        vals.append(mv)
        picks.append(jnp.max(jnp.where(iot == am, cidx, -1),
                             axis=1, keepdims=True))
        cur = jnp.where(iot == am, -jnp.inf, cur)
    sc = jnp.concatenate(vals, axis=1)
    idx = jnp.concatenate(picks, axis=1)
    e = jnp.exp(sc - jnp.max(sc, axis=1, keepdims=True))
    w_ref[...] = e / jnp.sum(e, axis=1, keepdims=True)
    idx_ref[...] = idx


def _pkm_topk(h, wk12):
    return pl.pallas_call(
        _pkm_topk_body,
        grid=(S // BP,),
        in_specs=[
            pl.BlockSpec((BP, D_MODEL), lambda i: (i, 0)),
            pl.BlockSpec((D_MODEL, 2 * N_KEYS), lambda i: (0, 0)),
        ],
        out_specs=[
            pl.BlockSpec((BP, KNN), lambda i: (i, 0)),
            pl.BlockSpec((BP, KNN), lambda i: (i, 0)),
        ],
        out_shape=[
            jax.ShapeDtypeStruct((S, KNN), jnp.float32),
            jax.ShapeDtypeStruct((S, KNN), jnp.int32),
        ],
    )(h, wk12)


def _pkm_gather_body(idx_hbm, w_hbm, y_hbm, values_hbm, out_hbm,
                     idx_v, w_v, acc_v, rows_a, rows_b, sem_a, sem_b):
    wid = lax.axis_index("s") * SC_NC + lax.axis_index("c")
    base = wid * TPW

    pltpu.sync_copy(idx_hbm.at[pl.ds(base * KNN, TPW * KNN)], idx_v)
    pltpu.sync_copy(w_hbm.at[pl.ds(base, TPW)], w_v)
    pltpu.sync_copy(y_hbm.at[pl.ds(base, TPW)], acc_v)

    def gather(t, rows, sem):
        return pltpu.make_async_copy(
            values_hbm.at[idx_v.at[pl.ds(t * KNN, KNN)]], rows, sem)

    def accumulate(t, rows):
        w0 = w_v[t, pl.ds(0, 16)]
        w1 = w_v[t, pl.ds(16, 16)]
        wk = [w0[k] for k in range(16)] + [w1[k] for k in range(16)]
        for d in range(D_MODEL // 16):
            sl = pl.ds(d * 16, 16)
            p = [acc_v[t, sl] + rows[0, sl] * wk[0]] + \
                [rows[j, sl] * wk[j] for j in range(1, 8)]
            for k in range(8, KNN):
                p[k % 8] = p[k % 8] + rows[k, sl] * wk[k]
            acc_v[t, sl] = (((p[0] + p[1]) + (p[2] + p[3])) +
                            ((p[4] + p[5]) + (p[6] + p[7])))

    gather(0, rows_a, sem_a).start()

    def pair(i, carry):
        t0 = 2 * i
        gather(t0 + 1, rows_b, sem_b).start()
        gather(t0, rows_a, sem_a).wait()
        accumulate(t0, rows_a)
        gather(lax.rem(t0 + 2, TPW), rows_a, sem_a).start()
        gather(t0 + 1, rows_b, sem_b).wait()
        accumulate(t0 + 1, rows_b)
        return carry

    lax.fori_loop(0, TPW // 2, pair, 0)
    gather(0, rows_a, sem_a).wait()

    pltpu.sync_copy(acc_v, out_hbm.at[pl.ds(base, TPW)])


@functools.cache
def _pkm_gather_kernel():
    return functools.partial(
        pl.kernel,
        mesh=plsc.VectorSubcoreMesh(core_axis_name="c", subcore_axis_name="s"),
        out_type=jax.ShapeDtypeStruct((B * S, D_MODEL), jnp.float32),
        scratch_types=[
            pltpu.VMEM((TPW * KNN,), jnp.int32),
            pltpu.VMEM((TPW, KNN), jnp.float32),
            pltpu.VMEM((TPW, D_MODEL), jnp.float32),
            pltpu.VMEM((KNN, D_MODEL), jnp.float32),
            pltpu.VMEM((KNN, D_MODEL), jnp.float32),
            pltpu.SemaphoreType.DMA,
            pltpu.SemaphoreType.DMA,
        ],
    )(_pkm_gather_body)


def _pkm_apply(idx, w, y, values):
    return _pkm_gather_kernel()(idx.reshape(B * S * KNN), w, y, values)


def kernel(src, Wq, Wk, Wv, Wo, Wr, g1, b1, g2, b2, k1, k2, values):
    x = src.reshape(S, D_MODEL)
    wqkv = jnp.concatenate([Wq, Wk, Wv], axis=1)
    qkv = _ln_qkv(x, g1.reshape(1, D_MODEL), b1.reshape(1, D_MODEL), wqkv)
    r = _rproj(_pos_enc_rev_padded(), Wr)

    def _heads(mat):
        n = mat.shape[0]
        return mat.reshape(n, NHEAD, HEAD).transpose(1, 0, 2)

    qh = _heads(qkv[:, :D_MODEL])
    kh = _heads(qkv[:, D_MODEL:2 * D_MODEL])
    vh = _heads(qkv[:, 2 * D_MODEL:])
    rh = _heads(r)
    attn = _flash_attn(qh, kh, vh, rh).transpose(1, 0, 2).reshape(S, D_MODEL)
    y, h2 = _oproj_ln(attn, x, g2.reshape(1, D_MODEL), b2.reshape(1, D_MODEL),
                      Wo)
    wk12 = jnp.zeros((D_MODEL, 2 * N_KEYS), jnp.float32)
    wk12 = wk12.at[:HALF, :N_KEYS].set(k1.T)
    wk12 = wk12.at[HALF:, N_KEYS:].set(k2.T)
    w, idx = _pkm_topk(h2, wk12)
    out = _pkm_apply(idx, w, y, values)
    return out.reshape(B, S, D_MODEL)


# speedup vs baseline: 1.3492x; 1.3492x over previous
import functools
import math

import numpy as np
import jax
import jax.numpy as jnp
from jax import lax
from jax.experimental import pallas as pl
from jax.experimental.pallas import tpu as pltpu
from jax.experimental.pallas import tpu_sc as plsc

B, S, D_MODEL, NHEAD = 1, 2048, 768, 12
HEAD = D_MODEL // NHEAD
N_KEYS, KNN = 256, 32
HALF = D_MODEL // 2
SCALE = 1.0 / math.sqrt(HEAD)

BQ = 256
NKB = S // BQ
BS = 512
BP = 256

SC_NC, SC_NS = 2, 16
SC_NW = SC_NC * SC_NS
TPW = (B * S) // SC_NW


def _pos_enc_rev_padded():
    inv_freq = 1.0 / (10000.0 ** (np.arange(0.0, D_MODEL, 2.0) / D_MODEL))
    pos = np.arange(-(S - 1), S, dtype=np.float64)[:, None] * inv_freq[None, :]
    pe = np.concatenate([np.sin(pos), np.cos(pos)], axis=-1)
    pe_rev = pe[::-1]
    out = np.zeros((2 * S, D_MODEL), dtype=np.float32)
    out[: 2 * S - 1] = pe_rev
    return jnp.asarray(out)


def _ln_qkv_body(x_ref, g_ref, b_ref, w_ref, qkv_ref):
    x = x_ref[...]
    m = jnp.mean(x, axis=1, keepdims=True)
    v = jnp.mean((x - m) ** 2, axis=1, keepdims=True)
    h = (x - m) * lax.rsqrt(v + 1e-5) * g_ref[...] + b_ref[...]
    qkv_ref[...] = jnp.dot(h, w_ref[...], preferred_element_type=jnp.float32)


def _ln_qkv(x, g, b, wqkv):
    return pl.pallas_call(
        _ln_qkv_body,
        grid=(S // BS,),
        in_specs=[
            pl.BlockSpec((BS, D_MODEL), lambda i: (i, 0)),
            pl.BlockSpec((1, D_MODEL), lambda i: (0, 0)),
            pl.BlockSpec((1, D_MODEL), lambda i: (0, 0)),
            pl.BlockSpec((D_MODEL, 3 * D_MODEL), lambda i: (0, 0)),
        ],
        out_specs=pl.BlockSpec((BS, 3 * D_MODEL), lambda i: (i, 0)),
        out_shape=jax.ShapeDtypeStruct((S, 3 * D_MODEL), jnp.float32),
    )(x, g, b, wqkv)


def _matmul_body(a_ref, w_ref, o_ref):
    o_ref[...] = jnp.dot(a_ref[...], w_ref[...],
                         preferred_element_type=jnp.float32)


def _rproj(pe_rev, wr):
    return pl.pallas_call(
        _matmul_body,
        grid=(2 * S // BS,),
        in_specs=[
            pl.BlockSpec((BS, D_MODEL), lambda i: (i, 0)),
            pl.BlockSpec((D_MODEL, D_MODEL), lambda i: (0, 0)),
        ],
        out_specs=pl.BlockSpec((BS, D_MODEL), lambda i: (i, 0)),
        out_shape=jax.ShapeDtypeStruct((2 * S, D_MODEL), jnp.float32),
    )(pe_rev, wr)


def _flash_body(q_ref, k_ref, v_ref, r_ref, o_ref):
    iq = pl.program_id(1)
    q = q_ref[0]
    wband = S + BQ
    s0 = (NKB - 1 - iq) * BQ
    rband = r_ref[0, pl.ds(s0, wband), :]
    tmp = lax.dot_general(q, rband, (((1,), (1,)), ((), ())),
                          preferred_element_type=jnp.float32)
    bd_full = pltpu.roll(tmp, shift=wband - BQ + 1, axis=1,
                         stride=1, stride_axis=0)
    m = jnp.full((BQ, 1), -jnp.inf, jnp.float32)
    l = jnp.zeros((BQ, 1), jnp.float32)
    acc = jnp.zeros((BQ, HEAD), jnp.float32)
    for jb in range(NKB):
        kblk = k_ref[0, pl.ds(jb * BQ, BQ), :]
        vblk = v_ref[0, pl.ds(jb * BQ, BQ), :]
        ac = lax.dot_general(q, kblk, (((1,), (1,)), ((), ())),
                             preferred_element_type=jnp.float32)
        s = (ac + bd_full[:, jb * BQ:(jb + 1) * BQ]) * SCALE
        mnew = jnp.maximum(m, jnp.max(s, axis=1, keepdims=True))
        p = jnp.exp(s - mnew)
        corr = jnp.exp(m - mnew)
        l = l * corr + jnp.sum(p, axis=1, keepdims=True)
        acc = acc * corr + jnp.dot(p, vblk, preferred_element_type=jnp.float32)
        m = mnew
    o_ref[0] = acc / l


def _flash_attn(qh, kh, vh, rh):
    return pl.pallas_call(
        _flash_body,
        grid=(NHEAD, S // BQ),
        in_specs=[
            pl.BlockSpec((1, BQ, HEAD), lambda h, i: (h, i, 0)),
            pl.BlockSpec((1, S, HEAD), lambda h, i: (h, 0, 0)),
            pl.BlockSpec((1, S, HEAD), lambda h, i: (h, 0, 0)),
            pl.BlockSpec((1, 2 * S, HEAD), lambda h, i: (h, 0, 0)),
        ],
        out_specs=pl.BlockSpec((1, BQ, HEAD), lambda h, i: (h, i, 0)),
        out_shape=jax.ShapeDtypeStruct((NHEAD, S, HEAD), jnp.float32),
    )(qh, kh, vh, rh)


def _oproj_ln_body(a_ref, x_ref, g_ref, b_ref, wo_ref, y_ref, h_ref):
    y = jnp.dot(a_ref[...], wo_ref[...],
                preferred_element_type=jnp.float32) + x_ref[...]
    y_ref[...] = y
    m = jnp.mean(y, axis=1, keepdims=True)
    v = jnp.mean((y - m) ** 2, axis=1, keepdims=True)
    h_ref[...] = (y - m) * lax.rsqrt(v + 1e-5) * g_ref[...] + b_ref[...]


def _oproj_ln(attn, x, g, b, wo):
    return pl.pallas_call(
        _oproj_ln_body,
        grid=(S // BS,),
        in_specs=[
            pl.BlockSpec((BS, D_MODEL), lambda i: (i, 0)),
            pl.BlockSpec((BS, D_MODEL), lambda i: (i, 0)),
            pl.BlockSpec((1, D_MODEL), lambda i: (0, 0)),
            pl.BlockSpec((1, D_MODEL), lambda i: (0, 0)),
            pl.BlockSpec((D_MODEL, D_MODEL), lambda i: (0, 0)),
        ],
        out_specs=[
            pl.BlockSpec((BS, D_MODEL), lambda i: (i, 0)),
            pl.BlockSpec((BS, D_MODEL), lambda i: (i, 0)),
        ],
        out_shape=[
            jax.ShapeDtypeStruct((S, D_MODEL), jnp.float32),
            jax.ShapeDtypeStruct((S, D_MODEL), jnp.float32),
        ],
    )(attn, x, g, b, wo)


def _monotone(x):
    u = lax.bitcast_convert_type(x, jnp.int32)
    return jnp.where(u >= 0, u, u ^ 0x7FFFFFFF)


def _unmonotone(k):
    b = jnp.where(k >= 0, k, k ^ 0x7FFFFFFF)
    return lax.bitcast_convert_type(b, jnp.float32)


def _top_k_packed(keys, k):
    outs = []
    cur = keys
    for _ in range(k):
        mv = jnp.max(cur, axis=1, keepdims=True)
        outs.append(mv)
        cur = jnp.where(cur == mv, jnp.int32(-2**31), cur)
    return jnp.concatenate(outs, axis=1)


def _pkm_topk_body(h_ref, wk_ref, w_ref, idx_ref):
    s = jnp.dot(h_ref[...], wk_ref[...], preferred_element_type=jnp.float32)
    pos = lax.broadcasted_iota(jnp.int32, (BP, N_KEYS), 1)
    m1 = _top_k_packed((_monotone(s[:, :N_KEYS]) & ~0xFF) | (255 - pos), KNN)
    m2 = _top_k_packed((_monotone(s[:, N_KEYS:]) & ~0xFF) | (255 - pos), KNN)
    i1 = 255 - (m1 & 255)
    i2 = 255 - (m2 & 255)
    sc1 = _unmonotone(m1 & ~0xFF)
    sc2 = _unmonotone(m2 & ~0xFF)
    parts = []
    for a in range(KNN):
        comb = sc1[:, a:a + 1] + sc2
        cidx = i1[:, a:a + 1] * N_KEYS + i2
        parts.append((_monotone(comb) & ~0xFFFF) | cidx)
    m = _top_k_packed(jnp.concatenate(parts, axis=1), KNN)
    idx = m & 0xFFFF
    sc = _unmonotone(m & ~0xFFFF)
    e = jnp.exp(sc - jnp.max(sc, axis=1, keepdims=True))
    w_ref[...] = e / jnp.sum(e, axis=1, keepdims=True)
    idx_ref[...] = idx


def _pkm_topk(h, wk12):
    return pl.pallas_call(
        _pkm_topk_body,
        grid=(S // BP,),
        in_specs=[
            pl.BlockSpec((BP, D_MODEL), lambda i: (i, 0)),
            pl.BlockSpec((D_MODEL, 2 * N_KEYS), lambda i: (0, 0)),
        ],
        out_specs=[
            pl.BlockSpec((BP, KNN), lambda i: (i, 0)),
            pl.BlockSpec((BP, KNN), lambda i: (i, 0)),
        ],
        out_shape=[
            jax.ShapeDtypeStruct((S, KNN), jnp.float32),
            jax.ShapeDtypeStruct((S, KNN), jnp.int32),
        ],
    )(h, wk12)


def _pkm_gather_body(idx_hbm, w_hbm, y_hbm, values_hbm, out_hbm,
                     idx_v, w_v, acc_v, rows_a, rows_b, sem_a, sem_b):
    wid = lax.axis_index("s") * SC_NC + lax.axis_index("c")
    base = wid * TPW

    pltpu.sync_copy(idx_hbm.at[pl.ds(base * KNN, TPW * KNN)], idx_v)
    pltpu.sync_copy(w_hbm.at[pl.ds(base, TPW)], w_v)
    pltpu.sync_copy(y_hbm.at[pl.ds(base, TPW)], acc_v)

    def gather(t, rows, sem):
        return pltpu.make_async_copy(
            values_hbm.at[idx_v.at[pl.ds(t * KNN, KNN)]], rows, sem)

    def accumulate(t, rows):
        w0 = w_v[t, pl.ds(0, 16)]
        w1 = w_v[t, pl.ds(16, 16)]
        wk = [w0[k] for k in range(16)] + [w1[k] for k in range(16)]
        for d in range(D_MODEL // 16):
            sl = pl.ds(d * 16, 16)
            p = [acc_v[t, sl] + rows[0, sl] * wk[0]] + \
                [rows[j, sl] * wk[j] for j in (1, 2, 3)]
            for k in range(4, KNN):
                p[k % 4] = p[k % 4] + rows[k, sl] * wk[k]
            acc_v[t, sl] = (p[0] + p[1]) + (p[2] + p[3])

    gather(0, rows_a, sem_a).start()

    def pair(i, carry):
        t0 = 2 * i
        gather(t0 + 1, rows_b, sem_b).start()
        gather(t0, rows_a, sem_a).wait()
        accumulate(t0, rows_a)
        gather(lax.rem(t0 + 2, TPW), rows_a, sem_a).start()
        gather(t0 + 1, rows_b, sem_b).wait()
        accumulate(t0 + 1, rows_b)
        return carry

    lax.fori_loop(0, TPW // 2, pair, 0)
    gather(0, rows_a, sem_a).wait()

    pltpu.sync_copy(acc_v, out_hbm.at[pl.ds(base, TPW)])


@functools.cache
def _pkm_gather_kernel():
    return functools.partial(
        pl.kernel,
        mesh=plsc.VectorSubcoreMesh(core_axis_name="c", subcore_axis_name="s"),
        out_type=jax.ShapeDtypeStruct((B * S, D_MODEL), jnp.float32),
        scratch_types=[
            pltpu.VMEM((TPW * KNN,), jnp.int32),
            pltpu.VMEM((TPW, KNN), jnp.float32),
            pltpu.VMEM((TPW, D_MODEL), jnp.float32),
            pltpu.VMEM((KNN, D_MODEL), jnp.float32),
            pltpu.VMEM((KNN, D_MODEL), jnp.float32),
            pltpu.SemaphoreType.DMA,
            pltpu.SemaphoreType.DMA,
        ],
    )(_pkm_gather_body)


def _pkm_apply(idx, w, y, values):
    return _pkm_gather_kernel()(idx.reshape(B * S * KNN), w, y, values)


def kernel(src, Wq, Wk, Wv, Wo, Wr, g1, b1, g2, b2, k1, k2, values):
    x = src.reshape(S, D_MODEL)
    wqkv = jnp.concatenate([Wq, Wk, Wv], axis=1)
    qkv = _ln_qkv(x, g1.reshape(1, D_MODEL), b1.reshape(1, D_MODEL), wqkv)
    r = _rproj(_pos_enc_rev_padded(), Wr)

    def _heads(mat):
        n = mat.shape[0]
        return mat.reshape(n, NHEAD, HEAD).transpose(1, 0, 2)

    qh = _heads(qkv[:, :D_MODEL])
    kh = _heads(qkv[:, D_MODEL:2 * D_MODEL])
    vh = _heads(qkv[:, 2 * D_MODEL:])
    rh = _heads(r)
    attn = _flash_attn(qh, kh, vh, rh).transpose(1, 0, 2).reshape(S, D_MODEL)
    y, h2 = _oproj_ln(attn, x, g2.reshape(1, D_MODEL), b2.reshape(1, D_MODEL),
                      Wo)
    wk12 = jnp.zeros((D_MODEL, 2 * N_KEYS), jnp.float32)
    wk12 = wk12.at[:HALF, :N_KEYS].set(k1.T)
    wk12 = wk12.at[HALF:, N_KEYS:].set(k2.T)
    w, idx = _pkm_topk(h2, wk12)
    out = _pkm_apply(idx, w, y, values)
    return out.reshape(B, S, D_MODEL)


# speedup vs baseline: 1.3526x; 1.0025x over previous
import functools
import math

import numpy as np
import jax
import jax.numpy as jnp
from jax import lax
from jax.experimental import pallas as pl
from jax.experimental.pallas import tpu as pltpu
from jax.experimental.pallas import tpu_sc as plsc

B, S, D_MODEL, NHEAD = 1, 2048, 768, 12
HEAD = D_MODEL // NHEAD
N_KEYS, KNN = 256, 32
HALF = D_MODEL // 2
SCALE = 1.0 / math.sqrt(HEAD)

BQ = 256
NKB = S // BQ
BS = 512
BP = 256

SC_NC, SC_NS = 2, 16
SC_NW = SC_NC * SC_NS
TPW = (B * S) // SC_NW


def _pos_enc_rev_padded():
    inv_freq = 1.0 / (10000.0 ** (np.arange(0.0, D_MODEL, 2.0) / D_MODEL))
    pos = np.arange(-(S - 1), S, dtype=np.float64)[:, None] * inv_freq[None, :]
    pe = np.concatenate([np.sin(pos), np.cos(pos)], axis=-1)
    pe_rev = pe[::-1]
    out = np.zeros((2 * S, D_MODEL), dtype=np.float32)
    out[: 2 * S - 1] = pe_rev
    return jnp.asarray(out)


def _ln_qkv_body(x_ref, g_ref, b_ref, w_ref, qkv_ref):
    x = x_ref[...]
    m = jnp.mean(x, axis=1, keepdims=True)
    v = jnp.mean((x - m) ** 2, axis=1, keepdims=True)
    h = (x - m) * lax.rsqrt(v + 1e-5) * g_ref[...] + b_ref[...]
    qkv_ref[...] = jnp.dot(h, w_ref[...], preferred_element_type=jnp.float32)


def _ln_qkv(x, g, b, wqkv):
    return pl.pallas_call(
        _ln_qkv_body,
        grid=(S // BS,),
        in_specs=[
            pl.BlockSpec((BS, D_MODEL), lambda i: (i, 0)),
            pl.BlockSpec((1, D_MODEL), lambda i: (0, 0)),
            pl.BlockSpec((1, D_MODEL), lambda i: (0, 0)),
            pl.BlockSpec((D_MODEL, 3 * D_MODEL), lambda i: (0, 0)),
        ],
        out_specs=pl.BlockSpec((BS, 3 * D_MODEL), lambda i: (i, 0)),
        out_shape=jax.ShapeDtypeStruct((S, 3 * D_MODEL), jnp.float32),
    )(x, g, b, wqkv)


def _matmul_body(a_ref, w_ref, o_ref):
    o_ref[...] = jnp.dot(a_ref[...], w_ref[...],
                         preferred_element_type=jnp.float32)


def _rproj(pe_rev, wr):
    return pl.pallas_call(
        _matmul_body,
        grid=(2 * S // BS,),
        in_specs=[
            pl.BlockSpec((BS, D_MODEL), lambda i: (i, 0)),
            pl.BlockSpec((D_MODEL, D_MODEL), lambda i: (0, 0)),
        ],
        out_specs=pl.BlockSpec((BS, D_MODEL), lambda i: (i, 0)),
        out_shape=jax.ShapeDtypeStruct((2 * S, D_MODEL), jnp.float32),
    )(pe_rev, wr)


def _flash_body(q_ref, k_ref, v_ref, r_ref, o_ref):
    iq = pl.program_id(1)
    q = q_ref[0]
    wband = S + BQ
    s0 = (NKB - 1 - iq) * BQ
    rband = r_ref[0, pl.ds(s0, wband), :]
    tmp = lax.dot_general(q, rband, (((1,), (1,)), ((), ())),
                          preferred_element_type=jnp.float32)
    bd_full = pltpu.roll(tmp, shift=wband - BQ + 1, axis=1,
                         stride=1, stride_axis=0)
    m = jnp.full((BQ, 1), -jnp.inf, jnp.float32)
    l = jnp.zeros((BQ, 1), jnp.float32)
    acc = jnp.zeros((BQ, HEAD), jnp.float32)
    for jb in range(NKB):
        kblk = k_ref[0, pl.ds(jb * BQ, BQ), :]
        vblk = v_ref[0, pl.ds(jb * BQ, BQ), :]
        ac = lax.dot_general(q, kblk, (((1,), (1,)), ((), ())),
                             preferred_element_type=jnp.float32)
        s = (ac + bd_full[:, jb * BQ:(jb + 1) * BQ]) * SCALE
        mnew = jnp.maximum(m, jnp.max(s, axis=1, keepdims=True))
        p = jnp.exp(s - mnew)
        corr = jnp.exp(m - mnew)
        l = l * corr + jnp.sum(p, axis=1, keepdims=True)
        acc = acc * corr + jnp.dot(p, vblk, preferred_element_type=jnp.float32)
        m = mnew
    o_ref[0] = acc / l


def _flash_attn(qh, kh, vh, rh):
    return pl.pallas_call(
        _flash_body,
        grid=(NHEAD, S // BQ),
        in_specs=[
            pl.BlockSpec((1, BQ, HEAD), lambda h, i: (h, i, 0)),
            pl.BlockSpec((1, S, HEAD), lambda h, i: (h, 0, 0)),
            pl.BlockSpec((1, S, HEAD), lambda h, i: (h, 0, 0)),
            pl.BlockSpec((1, 2 * S, HEAD), lambda h, i: (h, 0, 0)),
        ],
        out_specs=pl.BlockSpec((1, BQ, HEAD), lambda h, i: (h, i, 0)),
        out_shape=jax.ShapeDtypeStruct((NHEAD, S, HEAD), jnp.float32),
    )(qh, kh, vh, rh)


def _oproj_ln_body(a_ref, x_ref, g_ref, b_ref, wo_ref, y_ref, h_ref):
    y = jnp.dot(a_ref[...], wo_ref[...],
                preferred_element_type=jnp.float32) + x_ref[...]
    y_ref[...] = y
    m = jnp.mean(y, axis=1, keepdims=True)
    v = jnp.mean((y - m) ** 2, axis=1, keepdims=True)
    h_ref[...] = (y - m) * lax.rsqrt(v + 1e-5) * g_ref[...] + b_ref[...]


def _oproj_ln(attn, x, g, b, wo):
    return pl.pallas_call(
        _oproj_ln_body,
        grid=(S // BS,),
        in_specs=[
            pl.BlockSpec((BS, D_MODEL), lambda i: (i, 0)),
            pl.BlockSpec((BS, D_MODEL), lambda i: (i, 0)),
            pl.BlockSpec((1, D_MODEL), lambda i: (0, 0)),
            pl.BlockSpec((1, D_MODEL), lambda i: (0, 0)),
            pl.BlockSpec((D_MODEL, D_MODEL), lambda i: (0, 0)),
        ],
        out_specs=[
            pl.BlockSpec((BS, D_MODEL), lambda i: (i, 0)),
            pl.BlockSpec((BS, D_MODEL), lambda i: (i, 0)),
        ],
        out_shape=[
            jax.ShapeDtypeStruct((S, D_MODEL), jnp.float32),
            jax.ShapeDtypeStruct((S, D_MODEL), jnp.float32),
        ],
    )(attn, x, g, b, wo)


def _monotone(x):
    u = lax.bitcast_convert_type(x, jnp.int32)
    return jnp.where(u >= 0, u, u ^ 0x7FFFFFFF)


def _unmonotone(k):
    b = jnp.where(k >= 0, k, k ^ 0x7FFFFFFF)
    return lax.bitcast_convert_type(b, jnp.float32)


def _top_k_packed(keys, k):
    outs = []
    cur = keys
    for _ in range(k):
        mv = jnp.max(cur, axis=1, keepdims=True)
        outs.append(mv)
        cur = jnp.where(cur == mv, jnp.int32(-2**31), cur)
    return jnp.concatenate(outs, axis=1)


def _pkm_topk_body(h_ref, wk_ref, w_ref, idx_ref):
    s = jnp.dot(h_ref[...], wk_ref[...], preferred_element_type=jnp.float32)
    pos = lax.broadcasted_iota(jnp.int32, (BP, N_KEYS), 1)
    m1 = _top_k_packed((_monotone(s[:, :N_KEYS]) & ~0xFF) | (255 - pos), KNN)
    m2 = _top_k_packed((_monotone(s[:, N_KEYS:]) & ~0xFF) | (255 - pos), KNN)
    i1 = 255 - (m1 & 255)
    i2 = 255 - (m2 & 255)
    sc1 = _unmonotone(m1 & ~0xFF)
    sc2 = _unmonotone(m2 & ~0xFF)
    parts = []
    for a in range(KNN):
        comb = sc1[:, a:a + 1] + sc2
        cidx = i1[:, a:a + 1] * N_KEYS + i2
        parts.append((_monotone(comb) & ~0xFFFF) | cidx)
    m = _top_k_packed(jnp.concatenate(parts, axis=1), KNN)
    idx = m & 0xFFFF
    sc = _unmonotone(m & ~0xFFFF)
    e = jnp.exp(sc - jnp.max(sc, axis=1, keepdims=True))
    w_ref[...] = e / jnp.sum(e, axis=1, keepdims=True)
    idx_ref[...] = idx


def _pkm_topk(h, wk12):
    return pl.pallas_call(
        _pkm_topk_body,
        grid=(S // BP,),
        in_specs=[
            pl.BlockSpec((BP, D_MODEL), lambda i: (i, 0)),
            pl.BlockSpec((D_MODEL, 2 * N_KEYS), lambda i: (0, 0)),
        ],
        out_specs=[
            pl.BlockSpec((BP, KNN), lambda i: (i, 0)),
            pl.BlockSpec((BP, KNN), lambda i: (i, 0)),
        ],
        out_shape=[
            jax.ShapeDtypeStruct((S, KNN), jnp.float32),
            jax.ShapeDtypeStruct((S, KNN), jnp.int32),
        ],
    )(h, wk12)


def _pkm_gather_body(idx_hbm, w_hbm, y_hbm, values_hbm, out_hbm,
                     idx_v, w_v, acc_v, rows_a, rows_b, sem_a, sem_b):
    wid = lax.axis_index("s") * SC_NC + lax.axis_index("c")
    base = wid * TPW

    pltpu.sync_copy(idx_hbm.at[pl.ds(base * KNN, TPW * KNN)], idx_v)
    pltpu.sync_copy(w_hbm.at[pl.ds(base, TPW)], w_v)
    pltpu.sync_copy(y_hbm.at[pl.ds(base, TPW)], acc_v)

    def gather(t, rows, sem):
        return pltpu.make_async_copy(
            values_hbm.at[idx_v.at[pl.ds(t * KNN, KNN)]], rows, sem)

    def accumulate(t, rows):
        w0 = w_v[t, pl.ds(0, 16)]
        w1 = w_v[t, pl.ds(16, 16)]
        wk = [w0[k] for k in range(16)] + [w1[k] for k in range(16)]
        for d in range(D_MODEL // 16):
            sl = pl.ds(d * 16, 16)
            p = [acc_v[t, sl] + rows[0, sl] * wk[0]] + \
                [rows[j, sl] * wk[j] for j in (1, 2, 3)]
            for k in range(4, KNN):
                p[k % 4] = p[k % 4] + rows[k, sl] * wk[k]
            acc_v[t, sl] = (p[0] + p[1]) + (p[2] + p[3])

    gather(0, rows_a, sem_a).start()
    gather(1, rows_b, sem_b).start()

    def pair(i, carry):
        t0 = 2 * i
        gather(t0, rows_a, sem_a).wait()
        accumulate(t0, rows_a)
        gather(lax.rem(t0 + 2, TPW), rows_a, sem_a).start()
        gather(t0 + 1, rows_b, sem_b).wait()
        accumulate(t0 + 1, rows_b)
        gather(lax.rem(t0 + 3, TPW), rows_b, sem_b).start()
        return carry

    lax.fori_loop(0, TPW // 2, pair, 0)
    gather(0, rows_a, sem_a).wait()
    gather(1, rows_b, sem_b).wait()

    pltpu.sync_copy(acc_v, out_hbm.at[pl.ds(base, TPW)])


@functools.cache
def _pkm_gather_kernel():
    return functools.partial(
        pl.kernel,
        mesh=plsc.VectorSubcoreMesh(core_axis_name="c", subcore_axis_name="s"),
        out_type=jax.ShapeDtypeStruct((B * S, D_MODEL), jnp.float32),
        scratch_types=[
            pltpu.VMEM((TPW * KNN,), jnp.int32),
            pltpu.VMEM((TPW, KNN), jnp.float32),
            pltpu.VMEM((TPW, D_MODEL), jnp.float32),
            pltpu.VMEM((KNN, D_MODEL), jnp.float32),
            pltpu.VMEM((KNN, D_MODEL), jnp.float32),
            pltpu.SemaphoreType.DMA,
            pltpu.SemaphoreType.DMA,
        ],
    )(_pkm_gather_body)


def _pkm_apply(idx, w, y, values):
    return _pkm_gather_kernel()(idx.reshape(B * S * KNN), w, y, values)


def kernel(src, Wq, Wk, Wv, Wo, Wr, g1, b1, g2, b2, k1, k2, values):
    x = src.reshape(S, D_MODEL)
    wqkv = jnp.concatenate([Wq, Wk, Wv], axis=1)
    qkv = _ln_qkv(x, g1.reshape(1, D_MODEL), b1.reshape(1, D_MODEL), wqkv)
    r = _rproj(_pos_enc_rev_padded(), Wr)

    def _heads(mat):
        n = mat.shape[0]
        return mat.reshape(n, NHEAD, HEAD).transpose(1, 0, 2)

    qh = _heads(qkv[:, :D_MODEL])
    kh = _heads(qkv[:, D_MODEL:2 * D_MODEL])
    vh = _heads(qkv[:, 2 * D_MODEL:])
    rh = _heads(r)
    attn = _flash_attn(qh, kh, vh, rh).transpose(1, 0, 2).reshape(S, D_MODEL)
    y, h2 = _oproj_ln(attn, x, g2.reshape(1, D_MODEL), b2.reshape(1, D_MODEL),
                      Wo)
    wk12 = jnp.zeros((D_MODEL, 2 * N_KEYS), jnp.float32)
    wk12 = wk12.at[:HALF, :N_KEYS].set(k1.T)
    wk12 = wk12.at[HALF:, N_KEYS:].set(k2.T)
    w, idx = _pkm_topk(h2, wk12)
    out = _pkm_apply(idx, w, y, values)
    return out.reshape(B, S, D_MODEL)


# speedup vs baseline: 1.4462x; 1.0692x over previous
import functools
import math

import numpy as np
import jax
import jax.numpy as jnp
from jax import lax
from jax.experimental import pallas as pl
from jax.experimental.pallas import tpu as pltpu
from jax.experimental.pallas import tpu_sc as plsc

B, S, D_MODEL, NHEAD = 1, 2048, 768, 12
HEAD = D_MODEL // NHEAD
N_KEYS, KNN = 256, 32
HALF = D_MODEL // 2
SCALE = 1.0 / math.sqrt(HEAD)

BQ = 256
NKB = S // BQ
BS = 512
BP = 256

SC_NC, SC_NS = 2, 16
SC_NW = SC_NC * SC_NS
TPW = (B * S) // SC_NW


def _pos_enc_rev_padded():
    inv_freq = 1.0 / (10000.0 ** (np.arange(0.0, D_MODEL, 2.0) / D_MODEL))
    pos = np.arange(-(S - 1), S, dtype=np.float64)[:, None] * inv_freq[None, :]
    pe = np.concatenate([np.sin(pos), np.cos(pos)], axis=-1)
    pe_rev = pe[::-1]
    out = np.zeros((2 * S, D_MODEL), dtype=np.float32)
    out[: 2 * S - 1] = pe_rev
    return jnp.asarray(out)


def _ln_qkv_body(x_ref, g_ref, b_ref, w_ref, qkv_ref):
    x = x_ref[...]
    m = jnp.mean(x, axis=1, keepdims=True)
    v = jnp.mean((x - m) ** 2, axis=1, keepdims=True)
    h = (x - m) * lax.rsqrt(v + 1e-5) * g_ref[...] + b_ref[...]
    qkv_ref[...] = jnp.dot(h, w_ref[...], preferred_element_type=jnp.float32)


def _ln_qkv(x, g, b, wqkv):
    return pl.pallas_call(
        _ln_qkv_body,
        grid=(S // BS,),
        in_specs=[
            pl.BlockSpec((BS, D_MODEL), lambda i: (i, 0)),
            pl.BlockSpec((1, D_MODEL), lambda i: (0, 0)),
            pl.BlockSpec((1, D_MODEL), lambda i: (0, 0)),
            pl.BlockSpec((D_MODEL, 3 * D_MODEL), lambda i: (0, 0)),
        ],
        out_specs=pl.BlockSpec((BS, 3 * D_MODEL), lambda i: (i, 0)),
        out_shape=jax.ShapeDtypeStruct((S, 3 * D_MODEL), jnp.float32),
    )(x, g, b, wqkv)


def _matmul_body(a_ref, w_ref, o_ref):
    o_ref[...] = jnp.dot(a_ref[...], w_ref[...],
                         preferred_element_type=jnp.float32)


def _rproj(pe_rev, wr):
    return pl.pallas_call(
        _matmul_body,
        grid=(2 * S // BS,),
        in_specs=[
            pl.BlockSpec((BS, D_MODEL), lambda i: (i, 0)),
            pl.BlockSpec((D_MODEL, D_MODEL), lambda i: (0, 0)),
        ],
        out_specs=pl.BlockSpec((BS, D_MODEL), lambda i: (i, 0)),
        out_shape=jax.ShapeDtypeStruct((2 * S, D_MODEL), jnp.float32),
    )(pe_rev, wr)


def _flash_body(q_ref, k_ref, v_ref, r_ref, o_ref):
    iq = pl.program_id(1)
    q = q_ref[0]
    wband = S + BQ
    s0 = (NKB - 1 - iq) * BQ
    rband = r_ref[0, pl.ds(s0, wband), :]
    tmp = lax.dot_general(q, rband, (((1,), (1,)), ((), ())),
                          preferred_element_type=jnp.float32)
    bd = pltpu.roll(tmp, shift=wband - BQ + 1, axis=1,
                    stride=1, stride_axis=0)[:, :S]
    ac = lax.dot_general(q, k_ref[0], (((1,), (1,)), ((), ())),
                         preferred_element_type=jnp.float32)
    s = (ac + bd) * SCALE
    m = jnp.max(s, axis=1, keepdims=True)
    p = jnp.exp(s - m)
    l = jnp.sum(p, axis=1, keepdims=True)
    o_ref[0] = jnp.dot(p, v_ref[0],
                       preferred_element_type=jnp.float32) / l


def _flash_attn(qh, kh, vh, rh):
    return pl.pallas_call(
        _flash_body,
        grid=(NHEAD, S // BQ),
        in_specs=[
            pl.BlockSpec((1, BQ, HEAD), lambda h, i: (h, i, 0)),
            pl.BlockSpec((1, S, HEAD), lambda h, i: (h, 0, 0)),
            pl.BlockSpec((1, S, HEAD), lambda h, i: (h, 0, 0)),
            pl.BlockSpec((1, 2 * S, HEAD), lambda h, i: (h, 0, 0)),
        ],
        out_specs=pl.BlockSpec((1, BQ, HEAD), lambda h, i: (h, i, 0)),
        out_shape=jax.ShapeDtypeStruct((NHEAD, S, HEAD), jnp.float32),
    )(qh, kh, vh, rh)


def _oproj_ln_body(a_ref, x_ref, g_ref, b_ref, wo_ref, y_ref, h_ref):
    y = jnp.dot(a_ref[...], wo_ref[...],
                preferred_element_type=jnp.float32) + x_ref[...]
    y_ref[...] = y
    m = jnp.mean(y, axis=1, keepdims=True)
    v = jnp.mean((y - m) ** 2, axis=1, keepdims=True)
    h_ref[...] = (y - m) * lax.rsqrt(v + 1e-5) * g_ref[...] + b_ref[...]


def _oproj_ln(attn, x, g, b, wo):
    return pl.pallas_call(
        _oproj_ln_body,
        grid=(S // BS,),
        in_specs=[
            pl.BlockSpec((BS, D_MODEL), lambda i: (i, 0)),
            pl.BlockSpec((BS, D_MODEL), lambda i: (i, 0)),
            pl.BlockSpec((1, D_MODEL), lambda i: (0, 0)),
            pl.BlockSpec((1, D_MODEL), lambda i: (0, 0)),
            pl.BlockSpec((D_MODEL, D_MODEL), lambda i: (0, 0)),
        ],
        out_specs=[
            pl.BlockSpec((BS, D_MODEL), lambda i: (i, 0)),
            pl.BlockSpec((BS, D_MODEL), lambda i: (i, 0)),
        ],
        out_shape=[
            jax.ShapeDtypeStruct((S, D_MODEL), jnp.float32),
            jax.ShapeDtypeStruct((S, D_MODEL), jnp.float32),
        ],
    )(attn, x, g, b, wo)


def _monotone(x):
    u = lax.bitcast_convert_type(x, jnp.int32)
    return jnp.where(u >= 0, u, u ^ 0x7FFFFFFF)


def _unmonotone(k):
    b = jnp.where(k >= 0, k, k ^ 0x7FFFFFFF)
    return lax.bitcast_convert_type(b, jnp.float32)


def _top_k_packed(keys, k):
    outs = []
    cur = keys
    for _ in range(k):
        mv = jnp.max(cur, axis=1, keepdims=True)
        outs.append(mv)
        cur = jnp.where(cur == mv, jnp.int32(-2**31), cur)
    return jnp.concatenate(outs, axis=1)


def _pkm_topk_body(h_ref, wk_ref, w_ref, idx_ref):
    s = jnp.dot(h_ref[...], wk_ref[...], preferred_element_type=jnp.float32)
    pos = lax.broadcasted_iota(jnp.int32, (BP, N_KEYS), 1)
    m1 = _top_k_packed((_monotone(s[:, :N_KEYS]) & ~0xFF) | (255 - pos), KNN)
    m2 = _top_k_packed((_monotone(s[:, N_KEYS:]) & ~0xFF) | (255 - pos), KNN)
    i1 = 255 - (m1 & 255)
    i2 = 255 - (m2 & 255)
    sc1 = _unmonotone(m1 & ~0xFF)
    sc2 = _unmonotone(m2 & ~0xFF)
    parts = []
    for a in range(KNN):
        cnt = KNN // (a + 1)
        comb = sc1[:, a:a + 1] + sc2[:, :cnt]
        cidx = i1[:, a:a + 1] * N_KEYS + i2[:, :cnt]
        parts.append((_monotone(comb) & ~0xFFFF) | cidx)
    ncand = sum(KNN // (a + 1) for a in range(KNN))
    parts.append(jnp.full((BP, 128 - ncand), jnp.int32(-2**31)))
    m = _top_k_packed(jnp.concatenate(parts, axis=1), KNN)
    idx = m & 0xFFFF
    sc = _unmonotone(m & ~0xFFFF)
    e = jnp.exp(sc - jnp.max(sc, axis=1, keepdims=True))
    w_ref[...] = e / jnp.sum(e, axis=1, keepdims=True)
    idx_ref[...] = idx


def _pkm_topk(h, wk12):
    return pl.pallas_call(
        _pkm_topk_body,
        grid=(S // BP,),
        in_specs=[
            pl.BlockSpec((BP, D_MODEL), lambda i: (i, 0)),
            pl.BlockSpec((D_MODEL, 2 * N_KEYS), lambda i: (0, 0)),
        ],
        out_specs=[
            pl.BlockSpec((BP, KNN), lambda i: (i, 0)),
            pl.BlockSpec((BP, KNN), lambda i: (i, 0)),
        ],
        out_shape=[
            jax.ShapeDtypeStruct((S, KNN), jnp.float32),
            jax.ShapeDtypeStruct((S, KNN), jnp.int32),
        ],
    )(h, wk12)


def _pkm_gather_body(idx_hbm, w_hbm, y_hbm, values_hbm, out_hbm,
                     idx_v, w_v, acc_v, rows_a, rows_b, sem_a, sem_b):
    wid = lax.axis_index("s") * SC_NC + lax.axis_index("c")
    base = wid * TPW

    pltpu.sync_copy(idx_hbm.at[pl.ds(base * KNN, TPW * KNN)], idx_v)
    pltpu.sync_copy(w_hbm.at[pl.ds(base, TPW)], w_v)
    pltpu.sync_copy(y_hbm.at[pl.ds(base, TPW)], acc_v)

    def gather(t, rows, sem):
        return pltpu.make_async_copy(
            values_hbm.at[idx_v.at[pl.ds(t * KNN, KNN)]], rows, sem)

    def accumulate(t, rows):
        w0 = w_v[t, pl.ds(0, 16)]
        w1 = w_v[t, pl.ds(16, 16)]
        wk = [w0[k] for k in range(16)] + [w1[k] for k in range(16)]
        for d in range(D_MODEL // 16):
            sl = pl.ds(d * 16, 16)
            p = [acc_v[t, sl] + rows[0, sl] * wk[0]] + \
                [rows[j, sl] * wk[j] for j in (1, 2, 3)]
            for k in range(4, KNN):
                p[k % 4] = p[k % 4] + rows[k, sl] * wk[k]
            acc_v[t, sl] = (p[0] + p[1]) + (p[2] + p[3])

    gather(0, rows_a, sem_a).start()
    gather(1, rows_b, sem_b).start()

    def pair(i, carry):
        t0 = 2 * i
        gather(t0, rows_a, sem_a).wait()
        accumulate(t0, rows_a)
        gather(lax.rem(t0 + 2, TPW), rows_a, sem_a).start()
        gather(t0 + 1, rows_b, sem_b).wait()
        accumulate(t0 + 1, rows_b)
        gather(lax.rem(t0 + 3, TPW), rows_b, sem_b).start()
        return carry

    lax.fori_loop(0, TPW // 2, pair, 0)
    gather(0, rows_a, sem_a).wait()
    gather(1, rows_b, sem_b).wait()

    pltpu.sync_copy(acc_v, out_hbm.at[pl.ds(base, TPW)])


@functools.cache
def _pkm_gather_kernel():
    return functools.partial(
        pl.kernel,
        mesh=plsc.VectorSubcoreMesh(core_axis_name="c", subcore_axis_name="s"),
        out_type=jax.ShapeDtypeStruct((B * S, D_MODEL), jnp.float32),
        scratch_types=[
            pltpu.VMEM((TPW * KNN,), jnp.int32),
            pltpu.VMEM((TPW, KNN), jnp.float32),
            pltpu.VMEM((TPW, D_MODEL), jnp.float32),
            pltpu.VMEM((KNN, D_MODEL), jnp.float32),
            pltpu.VMEM((KNN, D_MODEL), jnp.float32),
            pltpu.SemaphoreType.DMA,
            pltpu.SemaphoreType.DMA,
        ],
    )(_pkm_gather_body)


def _pkm_apply(idx, w, y, values):
    return _pkm_gather_kernel()(idx.reshape(B * S * KNN), w, y, values)


def kernel(src, Wq, Wk, Wv, Wo, Wr, g1, b1, g2, b2, k1, k2, values):
    x = src.reshape(S, D_MODEL)
    wqkv = jnp.concatenate([Wq, Wk, Wv], axis=1)
    qkv = _ln_qkv(x, g1.reshape(1, D_MODEL), b1.reshape(1, D_MODEL), wqkv)
    r = _rproj(_pos_enc_rev_padded(), Wr)

    def _heads(mat):
        n = mat.shape[0]
        return mat.reshape(n, NHEAD, HEAD).transpose(1, 0, 2)

    qh = _heads(qkv[:, :D_MODEL])
    kh = _heads(qkv[:, D_MODEL:2 * D_MODEL])
    vh = _heads(qkv[:, 2 * D_MODEL:])
    rh = _heads(r)
    attn = _flash_attn(qh, kh, vh, rh).transpose(1, 0, 2).reshape(S, D_MODEL)
    y, h2 = _oproj_ln(attn, x, g2.reshape(1, D_MODEL), b2.reshape(1, D_MODEL),
                      Wo)
    wk12 = jnp.zeros((D_MODEL, 2 * N_KEYS), jnp.float32)
    wk12 = wk12.at[:HALF, :N_KEYS].set(k1.T)
    wk12 = wk12.at[HALF:, N_KEYS:].set(k2.T)
    w, idx = _pkm_topk(h2, wk12)
    out = _pkm_apply(idx, w, y, values)
    return out.reshape(B, S, D_MODEL)


# speedup vs baseline: 1.4521x; 1.0041x over previous
import functools
import math

import numpy as np
import jax
import jax.numpy as jnp
from jax import lax
from jax.experimental import pallas as pl
from jax.experimental.pallas import tpu as pltpu
from jax.experimental.pallas import tpu_sc as plsc

B, S, D_MODEL, NHEAD = 1, 2048, 768, 12
HEAD = D_MODEL // NHEAD
N_KEYS, KNN = 256, 32
HALF = D_MODEL // 2
SCALE = 1.0 / math.sqrt(HEAD)

BQ = 256
NKB = S // BQ
BS = 512
BP = 256

SC_NC, SC_NS = 2, 16
SC_NW = SC_NC * SC_NS
TPW = (B * S) // SC_NW


def _pos_enc_rev_padded():
    inv_freq = 1.0 / (10000.0 ** (np.arange(0.0, D_MODEL, 2.0) / D_MODEL))
    pos = np.arange(-(S - 1), S, dtype=np.float64)[:, None] * inv_freq[None, :]
    pe = np.concatenate([np.sin(pos), np.cos(pos)], axis=-1)
    pe_rev = pe[::-1]
    out = np.zeros((2 * S, D_MODEL), dtype=np.float32)
    out[: 2 * S - 1] = pe_rev
    return jnp.asarray(out)


def _ln_qkv_body(x_ref, g_ref, b_ref, w_ref, qkv_ref):
    x = x_ref[...]
    m = jnp.mean(x, axis=1, keepdims=True)
    v = jnp.mean((x - m) ** 2, axis=1, keepdims=True)
    h = (x - m) * lax.rsqrt(v + 1e-5) * g_ref[...] + b_ref[...]
    qkv_ref[...] = jnp.dot(h, w_ref[...], preferred_element_type=jnp.float32)


def _ln_qkv(x, g, b, wqkv):
    return pl.pallas_call(
        _ln_qkv_body,
        grid=(S // BS,),
        in_specs=[
            pl.BlockSpec((BS, D_MODEL), lambda i: (i, 0)),
            pl.BlockSpec((1, D_MODEL), lambda i: (0, 0)),
            pl.BlockSpec((1, D_MODEL), lambda i: (0, 0)),
            pl.BlockSpec((D_MODEL, 3 * D_MODEL), lambda i: (0, 0)),
        ],
        out_specs=pl.BlockSpec((BS, 3 * D_MODEL), lambda i: (i, 0)),
        out_shape=jax.ShapeDtypeStruct((S, 3 * D_MODEL), jnp.float32),
    )(x, g, b, wqkv)


def _matmul_body(a_ref, w_ref, o_ref):
    o_ref[...] = jnp.dot(a_ref[...], w_ref[...],
                         preferred_element_type=jnp.float32)


def _rproj(pe_rev, wr):
    return pl.pallas_call(
        _matmul_body,
        grid=(2 * S // BS,),
        in_specs=[
            pl.BlockSpec((BS, D_MODEL), lambda i: (i, 0)),
            pl.BlockSpec((D_MODEL, D_MODEL), lambda i: (0, 0)),
        ],
        out_specs=pl.BlockSpec((BS, D_MODEL), lambda i: (i, 0)),
        out_shape=jax.ShapeDtypeStruct((2 * S, D_MODEL), jnp.float32),
    )(pe_rev, wr)


def _flash_body(q_ref, k_ref, v_ref, r_ref, o_ref):
    iq = pl.program_id(1)
    q = q_ref[0]
    wband = S + BQ
    s0 = (NKB - 1 - iq) * BQ
    rband = r_ref[0, pl.ds(s0, wband), :]
    tmp = lax.dot_general(q, rband, (((1,), (1,)), ((), ())),
                          preferred_element_type=jnp.float32)
    bd = pltpu.roll(tmp, shift=wband - BQ + 1, axis=1,
                    stride=1, stride_axis=0)[:, :S]
    ac = lax.dot_general(q, k_ref[0], (((1,), (1,)), ((), ())),
                         preferred_element_type=jnp.float32)
    s = (ac + bd) * SCALE
    m = jnp.max(s, axis=1, keepdims=True)
    p = jnp.exp(s - m)
    l = jnp.sum(p, axis=1, keepdims=True)
    o_ref[0] = jnp.dot(p, v_ref[0],
                       preferred_element_type=jnp.float32) / l


def _flash_attn(qh, kh, vh, rh):
    return pl.pallas_call(
        _flash_body,
        grid=(NHEAD, S // BQ),
        in_specs=[
            pl.BlockSpec((1, BQ, HEAD), lambda h, i: (h, i, 0)),
            pl.BlockSpec((1, S, HEAD), lambda h, i: (h, 0, 0)),
            pl.BlockSpec((1, S, HEAD), lambda h, i: (h, 0, 0)),
            pl.BlockSpec((1, 2 * S, HEAD), lambda h, i: (h, 0, 0)),
        ],
        out_specs=pl.BlockSpec((1, BQ, HEAD), lambda h, i: (h, i, 0)),
        out_shape=jax.ShapeDtypeStruct((NHEAD, S, HEAD), jnp.float32),
    )(qh, kh, vh, rh)


def _monotone(x):
    u = lax.bitcast_convert_type(x, jnp.int32)
    return jnp.where(u >= 0, u, u ^ 0x7FFFFFFF)


def _unmonotone(k):
    b = jnp.where(k >= 0, k, k ^ 0x7FFFFFFF)
    return lax.bitcast_convert_type(b, jnp.float32)


def _top_k_packed(keys, k):
    outs = []
    cur = keys
    for _ in range(k):
        mv = jnp.max(cur, axis=1, keepdims=True)
        outs.append(mv)
        cur = jnp.where(cur == mv, jnp.int32(-2**31), cur)
    return jnp.concatenate(outs, axis=1)


def _tail_body(a_ref, x_ref, g_ref, b_ref, wo_ref, wk_ref,
               y_ref, w_ref, idx_ref):
    y = x_ref[...]
    for h in range(NHEAD):
        y = y + jnp.dot(a_ref[h], wo_ref[h],
                        preferred_element_type=jnp.float32)
    y_ref[...] = y
    mu = jnp.mean(y, axis=1, keepdims=True)
    var = jnp.mean((y - mu) ** 2, axis=1, keepdims=True)
    h2 = (y - mu) * lax.rsqrt(var + 1e-5) * g_ref[...] + b_ref[...]
    s = jnp.dot(h2, wk_ref[...], preferred_element_type=jnp.float32)
    pos = lax.broadcasted_iota(jnp.int32, (BP, N_KEYS), 1)
    m1 = _top_k_packed((_monotone(s[:, :N_KEYS]) & ~0xFF) | (255 - pos), KNN)
    m2 = _top_k_packed((_monotone(s[:, N_KEYS:]) & ~0xFF) | (255 - pos), KNN)
    i1 = 255 - (m1 & 255)
    i2 = 255 - (m2 & 255)
    sc1 = _unmonotone(m1 & ~0xFF)
    sc2 = _unmonotone(m2 & ~0xFF)
    parts = []
    for a in range(KNN):
        cnt = KNN // (a + 1)
        comb = sc1[:, a:a + 1] + sc2[:, :cnt]
        cidx = i1[:, a:a + 1] * N_KEYS + i2[:, :cnt]
        parts.append((_monotone(comb) & ~0xFFFF) | cidx)
    ncand = sum(KNN // (a + 1) for a in range(KNN))
    parts.append(jnp.full((BP, 128 - ncand), jnp.int32(-2**31)))
    m = _top_k_packed(jnp.concatenate(parts, axis=1), KNN)
    idx = m & 0xFFFF
    sc = _unmonotone(m & ~0xFFFF)
    e = jnp.exp(sc - jnp.max(sc, axis=1, keepdims=True))
    w_ref[...] = e / jnp.sum(e, axis=1, keepdims=True)
    idx_ref[...] = idx


def _tail(attn, x, g, b, wo, wk12):
    return pl.pallas_call(
        _tail_body,
        grid=(S // BP,),
        in_specs=[
            pl.BlockSpec((NHEAD, BP, HEAD), lambda i: (0, i, 0)),
            pl.BlockSpec((BP, D_MODEL), lambda i: (i, 0)),
            pl.BlockSpec((1, D_MODEL), lambda i: (0, 0)),
            pl.BlockSpec((1, D_MODEL), lambda i: (0, 0)),
            pl.BlockSpec((NHEAD, HEAD, D_MODEL), lambda i: (0, 0, 0)),
            pl.BlockSpec((D_MODEL, 2 * N_KEYS), lambda i: (0, 0)),
        ],
        out_specs=[
            pl.BlockSpec((BP, D_MODEL), lambda i: (i, 0)),
            pl.BlockSpec((BP, KNN), lambda i: (i, 0)),
            pl.BlockSpec((BP, KNN), lambda i: (i, 0)),
        ],
        out_shape=[
            jax.ShapeDtypeStruct((S, D_MODEL), jnp.float32),
            jax.ShapeDtypeStruct((S, KNN), jnp.float32),
            jax.ShapeDtypeStruct((S, KNN), jnp.int32),
        ],
    )(attn, x, g, b, wo, wk12)


def _pkm_gather_body(idx_hbm, w_hbm, y_hbm, values_hbm, out_hbm,
                     idx_v, w_v, acc_v, rows_a, rows_b, sem_a, sem_b):
    wid = lax.axis_index("s") * SC_NC + lax.axis_index("c")
    base = wid * TPW

    pltpu.sync_copy(idx_hbm.at[pl.ds(base * KNN, TPW * KNN)], idx_v)
    pltpu.sync_copy(w_hbm.at[pl.ds(base, TPW)], w_v)
    pltpu.sync_copy(y_hbm.at[pl.ds(base, TPW)], acc_v)

    def gather(t, rows, sem):
        return pltpu.make_async_copy(
            values_hbm.at[idx_v.at[pl.ds(t * KNN, KNN)]], rows, sem)

    def accumulate(t, rows):
        w0 = w_v[t, pl.ds(0, 16)]
        w1 = w_v[t, pl.ds(16, 16)]
        wk = [w0[k] for k in range(16)] + [w1[k] for k in range(16)]
        for d in range(D_MODEL // 16):
            sl = pl.ds(d * 16, 16)
            p = [acc_v[t, sl] + rows[0, sl] * wk[0]] + \
                [rows[j, sl] * wk[j] for j in (1, 2, 3)]
            for k in range(4, KNN):
                p[k % 4] = p[k % 4] + rows[k, sl] * wk[k]
            acc_v[t, sl] = (p[0] + p[1]) + (p[2] + p[3])

    gather(0, rows_a, sem_a).start()
    gather(1, rows_b, sem_b).start()

    def pair(i, carry):
        t0 = 2 * i
        gather(t0, rows_a, sem_a).wait()
        accumulate(t0, rows_a)
        gather(lax.rem(t0 + 2, TPW), rows_a, sem_a).start()
        gather(t0 + 1, rows_b, sem_b).wait()
        accumulate(t0 + 1, rows_b)
        gather(lax.rem(t0 + 3, TPW), rows_b, sem_b).start()
        return carry

    lax.fori_loop(0, TPW // 2, pair, 0)
    gather(0, rows_a, sem_a).wait()
    gather(1, rows_b, sem_b).wait()

    pltpu.sync_copy(acc_v, out_hbm.at[pl.ds(base, TPW)])


@functools.cache
def _pkm_gather_kernel():
    return functools.partial(
        pl.kernel,
        mesh=plsc.VectorSubcoreMesh(core_axis_name="c", subcore_axis_name="s"),
        out_type=jax.ShapeDtypeStruct((B * S, D_MODEL), jnp.float32),
        scratch_types=[
            pltpu.VMEM((TPW * KNN,), jnp.int32),
            pltpu.VMEM((TPW, KNN), jnp.float32),
            pltpu.VMEM((TPW, D_MODEL), jnp.float32),
            pltpu.VMEM((KNN, D_MODEL), jnp.float32),
            pltpu.VMEM((KNN, D_MODEL), jnp.float32),
            pltpu.SemaphoreType.DMA,
            pltpu.SemaphoreType.DMA,
        ],
    )(_pkm_gather_body)


def _pkm_apply(idx, w, y, values):
    return _pkm_gather_kernel()(idx.reshape(B * S * KNN), w, y, values)


def kernel(src, Wq, Wk, Wv, Wo, Wr, g1, b1, g2, b2, k1, k2, values):
    x = src.reshape(S, D_MODEL)
    wqkv = jnp.concatenate([Wq, Wk, Wv], axis=1)
    qkv = _ln_qkv(x, g1.reshape(1, D_MODEL), b1.reshape(1, D_MODEL), wqkv)
    r = _rproj(_pos_enc_rev_padded(), Wr)

    def _heads(mat):
        n = mat.shape[0]
        return mat.reshape(n, NHEAD, HEAD).transpose(1, 0, 2)

    qh = _heads(qkv[:, :D_MODEL])
    kh = _heads(qkv[:, D_MODEL:2 * D_MODEL])
    vh = _heads(qkv[:, 2 * D_MODEL:])
    rh = _heads(r)
    attn = _flash_attn(qh, kh, vh, rh)
    wk12 = jnp.zeros((D_MODEL, 2 * N_KEYS), jnp.float32)
    wk12 = wk12.at[:HALF, :N_KEYS].set(k1.T)
    wk12 = wk12.at[HALF:, N_KEYS:].set(k2.T)
    y, w, idx = _tail(attn, x, g2.reshape(1, D_MODEL),
                      b2.reshape(1, D_MODEL),
                      Wo.reshape(NHEAD, HEAD, D_MODEL), wk12)
    out = _pkm_apply(idx, w, y, values)
    return out.reshape(B, S, D_MODEL)
